# Initial kernel scaffold; baseline (speedup 1.0000x reference)
import jax
import jax.numpy as jnp
from jax.experimental import pallas as pl


def kernel(node_features, src_rel0, dst_rel0, src_rel1, dst_rel1, W_rel0, W_rel1, W_self, b_self):
    raise NotImplementedError("write your pallas kernel here")


# speedup vs baseline: 3.6141x; 3.6141x over previous
import functools

import jax
import jax.numpy as jnp
from jax import lax
from jax.experimental import pallas as pl
from jax.experimental.pallas import tpu as pltpu
from jax.experimental.pallas import tpu_sc as plsc


def _matmul_pad_body(x_ref, w_ref, o_ref):
    h = jnp.dot(x_ref[...], w_ref[0], preferred_element_type=jnp.float32)
    bm = h.shape[0]
    hp = jnp.concatenate([h, jnp.zeros((bm, 16), jnp.float32)], axis=1)
    col = lax.broadcasted_iota(jnp.int32, hp.shape, 1)
    o_ref[0] = jnp.where(col == 128, 1.0, hp)


def _finalize_body(x_ref, w_ref, b_ref, a0_ref, a1_ref, o_ref):
    s = jnp.dot(x_ref[...], w_ref[...], preferred_element_type=jnp.float32)
    s = s + b_ref[...]
    a0 = a0_ref[...]
    a1 = a1_ref[...]
    d0 = jnp.maximum(a0[:, 128:129], 1.0)
    d1 = jnp.maximum(a1[:, 128:129], 1.0)
    acc = a0[:, :128] / d0 + a1[:, :128] / d1 + s
    o_ref[...] = jnp.maximum(acc, 0.0)


def _make_sc_kernel(NP, DP, E_T, CH, R_T):
    mesh = plsc.VectorSubcoreMesh(core_axis_name="c", subcore_axis_name="s")

    @functools.partial(
        pl.kernel,
        out_type=(
            jax.ShapeDtypeStruct((NP, DP), jnp.float32),
            jax.ShapeDtypeStruct((NP, DP), jnp.float32),
        ),
        mesh=mesh,
        compiler_params=pltpu.CompilerParams(use_tc_tiling_on_sc=False),
        scratch_types=[
            pltpu.VMEM((CH,), jnp.int32),
            pltpu.VMEM((CH,), jnp.int32),
            pltpu.VMEM((CH, DP), jnp.float32),
            pltpu.VMEM_SHARED((NP, DP), jnp.float32),
            pltpu.SemaphoreType.DMA,
        ],
    )
    def sc_kernel(h0, h1, s0, d0, s1, d1, zrows, out0, out1,
                  src_v, dst_v, rows_v, agg_sh, sem):
        c = lax.axis_index("c")
        s = lax.axis_index("s")

        def process(h_hbm, src_hbm, dst_hbm, out_hbm):
            rbase = s * R_T
            pltpu.sync_copy(zrows.at[pl.ds(rbase, R_T)],
                            agg_sh.at[pl.ds(rbase, R_T)])
            plsc.subcore_barrier()
            ebase = s * E_T

            def body(j, carry):
                off = ebase + j * CH
                pltpu.sync_copy(src_hbm.at[pl.ds(off, CH)], src_v)
                pltpu.sync_copy(dst_hbm.at[pl.ds(off, CH)], dst_v)
                pltpu.async_copy(h_hbm.at[src_v], rows_v, sem).wait()
                pltpu.sync_copy(rows_v, agg_sh.at[dst_v], add=True)
                return carry

            lax.fori_loop(0, E_T // CH, body, 0)
            plsc.subcore_barrier()
            pltpu.sync_copy(agg_sh.at[pl.ds(rbase, R_T)],
                            out_hbm.at[pl.ds(rbase, R_T)])

        @pl.when(c == 0)
        def _():
            process(h0, s0, d0, out0)

        @pl.when(c == 1)
        def _():
            process(h1, s1, d1, out1)

    return sc_kernel


def kernel(node_features, src_rel0, dst_rel0, src_rel1, dst_rel1,
           W_rel0, W_rel1, W_self, b_self):
    N, D = node_features.shape
    D_OUT = W_rel0.shape[1]
    E = src_rel0.shape[0]
    DP = D_OUT + 16
    BM = 400
    NB = N // BM
    NT = 16
    E_T = E // NT
    CH = 80
    NP = 10240
    R_T = NP // NT

    Wstack = jnp.stack([W_rel0, W_rel1])
    H = pl.pallas_call(
        _matmul_pad_body,
        grid=(2, NB),
        in_specs=[
            pl.BlockSpec((BM, D), lambda r, i: (i, 0)),
            pl.BlockSpec((1, D, D_OUT), lambda r, i: (r, 0, 0)),
        ],
        out_specs=pl.BlockSpec((1, BM, DP), lambda r, i: (r, i, 0)),
        out_shape=jax.ShapeDtypeStruct((2, N, DP), jnp.float32),
    )(node_features, Wstack)

    zrows = jnp.zeros((NP, DP), jnp.float32)
    sc_fn = _make_sc_kernel(NP, DP, E_T, CH, R_T)
    agg0, agg1 = sc_fn(H[0], H[1], src_rel0, dst_rel0, src_rel1, dst_rel1,
                       zrows)

    b2 = b_self.reshape(1, D_OUT)
    out = pl.pallas_call(
        _finalize_body,
        grid=(NB,),
        in_specs=[
            pl.BlockSpec((BM, D), lambda i: (i, 0)),
            pl.BlockSpec((D, D_OUT), lambda i: (0, 0)),
            pl.BlockSpec((1, D_OUT), lambda i: (0, 0)),
            pl.BlockSpec((BM, DP), lambda i: (i, 0)),
            pl.BlockSpec((BM, DP), lambda i: (i, 0)),
        ],
        out_specs=pl.BlockSpec((BM, D_OUT), lambda i: (i, 0)),
        out_shape=jax.ShapeDtypeStruct((N, D_OUT), jnp.float32),
    )(node_features, W_self, b2, agg0, agg1)
    return out


# speedup vs baseline: 6.3566x; 1.7588x over previous
import functools

import jax
import jax.numpy as jnp
from jax import lax
from jax.experimental import pallas as pl
from jax.experimental.pallas import tpu as pltpu
from jax.experimental.pallas import tpu_sc as plsc


def _matmul_pad_body(x_ref, w_ref, o_ref):
    h = jnp.dot(x_ref[...], w_ref[0], preferred_element_type=jnp.float32)
    bm = h.shape[0]
    hp = jnp.concatenate([h, jnp.zeros((bm, 16), jnp.float32)], axis=1)
    col = lax.broadcasted_iota(jnp.int32, hp.shape, 1)
    o_ref[0] = jnp.where(col == 128, 1.0, hp)


def _finalize_body(x_ref, w_ref, b_ref, a0_ref, a1_ref, o_ref):
    s = jnp.dot(x_ref[...], w_ref[...], preferred_element_type=jnp.float32)
    s = s + b_ref[...]
    a0 = a0_ref[...]
    a1 = a1_ref[...]
    d0 = jnp.maximum(a0[:, 128:129], 1.0)
    d1 = jnp.maximum(a1[:, 128:129], 1.0)
    acc = a0[:, :128] / d0 + a1[:, :128] / d1 + s
    o_ref[...] = jnp.maximum(acc, 0.0)


def _make_sc_kernel(NP, DP, NCH, CH, R_T):
    mesh = plsc.VectorSubcoreMesh(core_axis_name="c", subcore_axis_name="s")

    @functools.partial(
        pl.kernel,
        out_type=(
            jax.ShapeDtypeStruct((NP, DP), jnp.float32),
            jax.ShapeDtypeStruct((NP, DP), jnp.float32),
        ),
        mesh=mesh,
        compiler_params=pltpu.CompilerParams(use_tc_tiling_on_sc=False),
        scratch_types=[
            pltpu.VMEM((NCH, CH), jnp.int32),
            pltpu.VMEM((CH,), jnp.int32),
            pltpu.VMEM((CH,), jnp.int32),
            pltpu.VMEM((CH, DP), jnp.float32),
            pltpu.VMEM((CH, DP), jnp.float32),
            pltpu.VMEM_SHARED((NP, DP), jnp.float32),
            pltpu.SemaphoreType.DMA,
            pltpu.SemaphoreType.DMA,
            pltpu.SemaphoreType.DMA,
            pltpu.SemaphoreType.DMA,
        ],
    )
    def sc_kernel(h0, h1, s0, d0, s1, d1, zrows, out0, out1,
                  src_v, dst_a, dst_b, rows_a, rows_b, agg_sh,
                  gsem_a, gsem_b, dsem_a, dsem_b):
        c = lax.axis_index("c")
        s = lax.axis_index("s")

        def process(h_hbm, src_hbm, dst_hbm, out_hbm):
            rbase = s * R_T
            pltpu.async_copy(src_hbm.at[s], src_v, gsem_a)
            pltpu.sync_copy(zrows.at[pl.ds(rbase, R_T)],
                            agg_sh.at[pl.ds(rbase, R_T)])
            pltpu.make_async_copy(src_hbm.at[s], src_v, gsem_a).wait()
            plsc.subcore_barrier()

            def gather_start(j, rows_v, sem):
                pltpu.async_copy(h_hbm.at[src_v.at[j]], rows_v, sem)

            def gather_wait(rows_v, sem):
                pltpu.make_async_copy(h_hbm.at[src_v.at[0]], rows_v,
                                      sem).wait()

            def dst_start(j, dst_v, sem):
                pltpu.async_copy(dst_hbm.at[s, j], dst_v, sem)

            def dst_wait(dst_v, sem):
                pltpu.make_async_copy(dst_hbm.at[s, 0], dst_v, sem).wait()

            def scatter(rows_v, dst_v):
                pltpu.sync_copy(rows_v, agg_sh.at[dst_v], add=True)

            gather_start(0, rows_a, gsem_a)
            dst_start(0, dst_a, dsem_a)

            def pair(g, carry):
                j = 2 * g

                @pl.when(j + 1 < NCH)
                def _():
                    gather_start(j + 1, rows_b, gsem_b)
                    dst_start(j + 1, dst_b, dsem_b)

                gather_wait(rows_a, gsem_a)
                dst_wait(dst_a, dsem_a)
                scatter(rows_a, dst_a)

                @pl.when(j + 2 < NCH)
                def _():
                    gather_start(j + 2, rows_a, gsem_a)
                    dst_start(j + 2, dst_a, dsem_a)

                @pl.when(j + 1 < NCH)
                def _():
                    gather_wait(rows_b, gsem_b)
                    dst_wait(dst_b, dsem_b)
                    scatter(rows_b, dst_b)

                return carry

            lax.fori_loop(0, (NCH + 1) // 2, pair, 0)
            plsc.subcore_barrier()
            pltpu.sync_copy(agg_sh.at[pl.ds(rbase, R_T)],
                            out_hbm.at[pl.ds(rbase, R_T)])

        @pl.when(c == 0)
        def _():
            process(h0, s0, d0, out0)

        @pl.when(c == 1)
        def _():
            process(h1, s1, d1, out1)

    return sc_kernel


def kernel(node_features, src_rel0, dst_rel0, src_rel1, dst_rel1,
           W_rel0, W_rel1, W_self, b_self):
    N, D = node_features.shape
    D_OUT = W_rel0.shape[1]
    E = src_rel0.shape[0]
    DP = D_OUT + 16
    BM = 400
    NB = N // BM
    NT = 16
    E_T = E // NT
    CH = 80
    NCH = E_T // CH
    NP = 10240
    R_T = NP // NT

    Wstack = jnp.stack([W_rel0, W_rel1])
    H = pl.pallas_call(
        _matmul_pad_body,
        grid=(2, NB),
        in_specs=[
            pl.BlockSpec((BM, D), lambda r, i: (i, 0)),
            pl.BlockSpec((1, D, D_OUT), lambda r, i: (r, 0, 0)),
        ],
        out_specs=pl.BlockSpec((1, BM, DP), lambda r, i: (r, i, 0)),
        out_shape=jax.ShapeDtypeStruct((2, N, DP), jnp.float32),
    )(node_features, Wstack)

    zrows = jnp.zeros((NP, DP), jnp.float32)
    sc_fn = _make_sc_kernel(NP, DP, NCH, CH, R_T)
    agg0, agg1 = sc_fn(H[0], H[1],
                       src_rel0.reshape(NT, NCH, CH),
                       dst_rel0.reshape(NT, NCH, CH),
                       src_rel1.reshape(NT, NCH, CH),
                       dst_rel1.reshape(NT, NCH, CH),
                       zrows)

    b2 = b_self.reshape(1, D_OUT)
    out = pl.pallas_call(
        _finalize_body,
        grid=(NB,),
        in_specs=[
            pl.BlockSpec((BM, D), lambda i: (i, 0)),
            pl.BlockSpec((D, D_OUT), lambda i: (0, 0)),
            pl.BlockSpec((1, D_OUT), lambda i: (0, 0)),
            pl.BlockSpec((BM, DP), lambda i: (i, 0)),
            pl.BlockSpec((BM, DP), lambda i: (i, 0)),
        ],
        out_specs=pl.BlockSpec((BM, D_OUT), lambda i: (i, 0)),
        out_shape=jax.ShapeDtypeStruct((N, D_OUT), jnp.float32),
    )(node_features, W_self, b2, agg0, agg1)
    return out


# speedup vs baseline: 7.1604x; 1.1265x over previous
import functools

import jax
import jax.numpy as jnp
from jax import lax
from jax.experimental import pallas as pl
from jax.experimental.pallas import tpu as pltpu
from jax.experimental.pallas import tpu_sc as plsc


def _matmuls_body(x_ref, w0_ref, w1_ref, ws_ref, b_ref,
                  h0_ref, h1_ref, s_ref):
    x = x_ref[...]

    def padded(w):
        h = jnp.dot(x, w, preferred_element_type=jnp.float32)
        hp = jnp.concatenate([h, jnp.zeros((h.shape[0], 16), jnp.float32)],
                             axis=1)
        col = lax.broadcasted_iota(jnp.int32, hp.shape, 1)
        return jnp.where(col == 128, 1.0, hp)

    h0_ref[...] = padded(w0_ref[...])
    h1_ref[...] = padded(w1_ref[...])
    s_ref[...] = (jnp.dot(x, ws_ref[...], preferred_element_type=jnp.float32)
                  + b_ref[...])


def _finalize_body(s_ref, a0_ref, a1_ref, o_ref):
    a0 = a0_ref[...]
    a1 = a1_ref[...]
    d0 = jnp.maximum(a0[:, 128:129], 1.0)
    d1 = jnp.maximum(a1[:, 128:129], 1.0)
    acc = a0[:, :128] / d0 + a1[:, :128] / d1 + s_ref[...]
    o_ref[...] = jnp.maximum(acc, 0.0)


def _make_sc_kernel(NP, DP, NCH, CH, R_T):
    mesh = plsc.VectorSubcoreMesh(core_axis_name="c", subcore_axis_name="s")

    @functools.partial(
        pl.kernel,
        out_type=(
            jax.ShapeDtypeStruct((NP, DP), jnp.float32),
            jax.ShapeDtypeStruct((NP, DP), jnp.float32),
        ),
        mesh=mesh,
        compiler_params=pltpu.CompilerParams(use_tc_tiling_on_sc=False),
        scratch_types=[
            pltpu.VMEM((NCH, CH), jnp.int32),
            pltpu.VMEM((CH,), jnp.int32),
            pltpu.VMEM((CH,), jnp.int32),
            pltpu.VMEM((CH, DP), jnp.float32),
            pltpu.VMEM((CH, DP), jnp.float32),
            pltpu.VMEM_SHARED((NP, DP), jnp.float32),
            pltpu.SemaphoreType.DMA,
            pltpu.SemaphoreType.DMA,
            pltpu.SemaphoreType.DMA,
            pltpu.SemaphoreType.DMA,
        ],
    )
    def sc_kernel(h0, h1, s0, d0, s1, d1, zrows, out0, out1,
                  src_v, dst_a, dst_b, rows_a, rows_b, agg_sh,
                  gsem_a, gsem_b, dsem_a, dsem_b):
        c = lax.axis_index("c")
        s = lax.axis_index("s")

        def process(h_hbm, src_hbm, dst_hbm, out_hbm):
            rbase = s * R_T
            pltpu.async_copy(src_hbm.at[s], src_v, gsem_a)
            pltpu.sync_copy(zrows.at[pl.ds(rbase, R_T)],
                            agg_sh.at[pl.ds(rbase, R_T)])
            pltpu.make_async_copy(src_hbm.at[s], src_v, gsem_a).wait()
            plsc.subcore_barrier()

            def gather_start(j, rows_v, sem):
                pltpu.async_copy(h_hbm.at[src_v.at[j]], rows_v, sem)

            def gather_wait(rows_v, sem):
                pltpu.make_async_copy(h_hbm.at[src_v.at[0]], rows_v,
                                      sem).wait()

            def dst_start(j, dst_v, sem):
                pltpu.async_copy(dst_hbm.at[s, j], dst_v, sem)

            def dst_wait(dst_v, sem):
                pltpu.make_async_copy(dst_hbm.at[s, 0], dst_v, sem).wait()

            def scatter(rows_v, dst_v):
                pltpu.sync_copy(rows_v, agg_sh.at[dst_v], add=True)

            gather_start(0, rows_a, gsem_a)
            dst_start(0, dst_a, dsem_a)

            def pair(g, carry):
                j = 2 * g

                @pl.when(j + 1 < NCH)
                def _():
                    gather_start(j + 1, rows_b, gsem_b)
                    dst_start(j + 1, dst_b, dsem_b)

                gather_wait(rows_a, gsem_a)
                dst_wait(dst_a, dsem_a)
                scatter(rows_a, dst_a)

                @pl.when(j + 2 < NCH)
                def _():
                    gather_start(j + 2, rows_a, gsem_a)
                    dst_start(j + 2, dst_a, dsem_a)

                @pl.when(j + 1 < NCH)
                def _():
                    gather_wait(rows_b, gsem_b)
                    dst_wait(dst_b, dsem_b)
                    scatter(rows_b, dst_b)

                return carry

            lax.fori_loop(0, (NCH + 1) // 2, pair, 0)
            plsc.subcore_barrier()
            pltpu.sync_copy(agg_sh.at[pl.ds(rbase, R_T)],
                            out_hbm.at[pl.ds(rbase, R_T)])

        @pl.when(c == 0)
        def _():
            process(h0, s0, d0, out0)

        @pl.when(c == 1)
        def _():
            process(h1, s1, d1, out1)

    return sc_kernel


def kernel(node_features, src_rel0, dst_rel0, src_rel1, dst_rel1,
           W_rel0, W_rel1, W_self, b_self):
    N, D = node_features.shape
    D_OUT = W_rel0.shape[1]
    E = src_rel0.shape[0]
    DP = D_OUT + 16
    BM = 400
    NB = N // BM
    NT = 16
    E_T = E // NT
    CH = 80
    NCH = E_T // CH
    NP = 10240
    R_T = NP // NT

    b2 = b_self.reshape(1, D_OUT)
    H0, H1, S = pl.pallas_call(
        _matmuls_body,
        grid=(NB,),
        in_specs=[
            pl.BlockSpec((BM, D), lambda i: (i, 0)),
            pl.BlockSpec((D, D_OUT), lambda i: (0, 0)),
            pl.BlockSpec((D, D_OUT), lambda i: (0, 0)),
            pl.BlockSpec((D, D_OUT), lambda i: (0, 0)),
            pl.BlockSpec((1, D_OUT), lambda i: (0, 0)),
        ],
        out_specs=[
            pl.BlockSpec((BM, DP), lambda i: (i, 0)),
            pl.BlockSpec((BM, DP), lambda i: (i, 0)),
            pl.BlockSpec((BM, D_OUT), lambda i: (i, 0)),
        ],
        out_shape=[
            jax.ShapeDtypeStruct((N, DP), jnp.float32),
            jax.ShapeDtypeStruct((N, DP), jnp.float32),
            jax.ShapeDtypeStruct((N, D_OUT), jnp.float32),
        ],
    )(node_features, W_rel0, W_rel1, W_self, b2)

    zrows = jnp.zeros((NP, DP), jnp.float32)
    sc_fn = _make_sc_kernel(NP, DP, NCH, CH, R_T)
    agg0, agg1 = sc_fn(H0, H1,
                       src_rel0.reshape(NT, NCH, CH),
                       dst_rel0.reshape(NT, NCH, CH),
                       src_rel1.reshape(NT, NCH, CH),
                       dst_rel1.reshape(NT, NCH, CH),
                       zrows)

    out = pl.pallas_call(
        _finalize_body,
        grid=(NB,),
        in_specs=[
            pl.BlockSpec((BM, D_OUT), lambda i: (i, 0)),
            pl.BlockSpec((BM, DP), lambda i: (i, 0)),
            pl.BlockSpec((BM, DP), lambda i: (i, 0)),
        ],
        out_specs=pl.BlockSpec((BM, D_OUT), lambda i: (i, 0)),
        out_shape=jax.ShapeDtypeStruct((N, D_OUT), jnp.float32),
    )(S, agg0, agg1)
    return out
